# Initial kernel scaffold; baseline (speedup 1.0000x reference)
import jax
import jax.numpy as jnp
from jax.experimental import pallas as pl


def kernel(input_features_in, label_mask):
    raise NotImplementedError("write your pallas kernel here")


# speedup vs baseline: 13.3995x; 13.3995x over previous
import functools

import jax
import jax.numpy as jnp
from jax import lax
from jax.experimental import pallas as pl
from jax.experimental.pallas import tpu as pltpu
from jax.experimental.pallas import tpu_sc as plsc

_NSEG = 1024
_DUMMY = _NSEG
_ACC_COLS = _NSEG + 16
_B, _C, _H, _W = 4, 96, 384, 384
_GROUPS = 8
_CPT = _C // _GROUPS
_ROWS_PER_CHUNK = 2
_CHUNK = _ROWS_PER_CHUNK * _W
_NCHUNKS = (_H - 2) // _ROWS_PER_CHUNK
_NVEC = _CHUNK // 16


def _body(x_hbm, lbl_hbm, bm_hbm, out_hbm,
          valbuf, lblbuf, bmbuf, acc, claim,
          sem_v0, sem_v1, sem_l0, sem_l1):
    wid = lax.axis_index("s") * 2 + lax.axis_index("c")
    b = wid // _GROUPS
    c0 = (wid % _GROUPS) * _CPT

    sem_v = (sem_v0, sem_v1)
    sem_l = (sem_l0, sem_l1)

    pltpu.sync_copy(bm_hbm, bmbuf)

    neg_inf = jnp.full((16,), -jnp.inf, jnp.float32)

    def init_body(i, _):
        for c in range(_CPT):
            acc[c, pl.ds(i * 16, 16)] = neg_inf
        return _

    lax.fori_loop(0, _ACC_COLS // 16, init_body, None)

    def issue(k, slot):
        off = (1 + _ROWS_PER_CHUNK * k) * _W
        pltpu.make_async_copy(
            x_hbm.at[b, pl.ds(c0, _CPT), pl.ds(off, _CHUNK)],
            valbuf.at[slot], sem_v[slot]).start()
        pltpu.make_async_copy(
            lbl_hbm.at[b, pl.ds(off, _CHUNK)],
            lblbuf.at[slot], sem_l[slot]).start()

    def wait(slot):
        pltpu.make_async_copy(
            x_hbm.at[b, pl.ds(c0, _CPT), pl.ds(0, _CHUNK)],
            valbuf.at[slot], sem_v[slot]).wait()
        pltpu.make_async_copy(
            lbl_hbm.at[b, pl.ds(0, _CHUNK)],
            lblbuf.at[slot], sem_l[slot]).wait()

    iota16 = lax.iota(jnp.int32, 16)
    dummy16 = jnp.full((16,), _DUMMY, jnp.int32)
    rowidx = [jnp.full((16,), c, jnp.int32) for c in range(_CPT)]

    ones16 = jnp.ones((16,), jnp.int32)
    zeros16 = jnp.zeros((16,), jnp.int32)

    def process(slot):
        def jbody(j, _):
            j16 = j * 16
            lbl = lblbuf[slot, pl.ds(j16, 16)]
            bm = bmbuf[pl.ds(j16, 16)]
            la = jnp.where(bm != 0, dummy16, lbl)
            vs = [valbuf[slot, c, pl.ds(j16, 16)] for c in range(_CPT)]

            def do_round(pend_m):
                plsc.store_scatter(claim, [la], iota16, mask=pend_m)
                got = plsc.load_gather(claim, [la])
                win = jnp.logical_and(pend_m, got == iota16)
                for c in range(_CPT):
                    cur = plsc.load_gather(acc, [rowidx[c], la])
                    plsc.store_scatter(acc, [rowidx[c], la],
                                       jnp.maximum(cur, vs[c]), mask=win)
                return jnp.logical_and(pend_m, jnp.logical_not(win))

            def rec(r, pend_m):
                if r >= 16:
                    return
                any_pending = jnp.max(jnp.where(pend_m, ones16, zeros16)) > 0

                @pl.when(any_pending)
                def _next_round():
                    rec(r + 1, do_round(pend_m))

            rec(1, do_round(iota16 >= 0))
            return _

        lax.fori_loop(0, _NVEC, jbody, None)

    issue(0, 0)

    def outer(i, _):
        k0 = 2 * i
        issue(k0 + 1, 1)
        wait(0)
        process(0)

        @pl.when(k0 + 2 < _NCHUNKS)
        def _issue_next():
            issue(k0 + 2, 0)

        wait(1)
        process(1)
        return _

    lax.fori_loop(0, _NCHUNKS // 2, outer, None)
    wait(0)
    process(0)

    for c in range(_CPT):
        pltpu.sync_copy(acc.at[c, pl.ds(0, _NSEG)], out_hbm.at[b, c0 + c])


@jax.jit
def kernel(input_features_in, label_mask):
    x = input_features_in.reshape(_B, _C, _H * _W)
    lbl = label_mask.reshape(_B, _H * _W)
    col = jnp.arange(_CHUNK, dtype=jnp.int32) % _W
    bm = ((col == 0) | (col == _W - 1)).astype(jnp.int32)

    mesh = plsc.VectorSubcoreMesh(core_axis_name="c", subcore_axis_name="s",
                                  num_cores=2, num_subcores=16)
    run = pl.kernel(
        _body,
        out_type=jax.ShapeDtypeStruct((_B, _C, _NSEG), jnp.float32),
        mesh=mesh,
        scratch_types=[
            pltpu.VMEM((2, _CPT, _CHUNK), jnp.float32),
            pltpu.VMEM((2, _CHUNK), jnp.int32),
            pltpu.VMEM((_CHUNK,), jnp.int32),
            pltpu.VMEM((_CPT, _ACC_COLS), jnp.float32),
            pltpu.VMEM((_ACC_COLS,), jnp.int32),
            pltpu.SemaphoreType.DMA,
            pltpu.SemaphoreType.DMA,
            pltpu.SemaphoreType.DMA,
            pltpu.SemaphoreType.DMA,
        ],
        compiler_params=pltpu.CompilerParams(use_tc_tiling_on_sc=False,
                                             needs_layout_passes=False),
    )
    return run(x, lbl, bm)
